# baseline (speedup 1.0000x reference)
import functools

import jax
import jax.numpy as jnp
from jax import lax
from jax.experimental import pallas as pl
from jax.experimental.pallas import tpu as pltpu
from jax.experimental.pallas import tpu_sc as plsc

N = 10000
E = 320000
D = 128
H = 128
DE = 16

NC = 2
NS = 16
NW = NC * NS
GW = 80
EPT = E // NW
CHUNKS = EPT // GW
RPT = 632
NP = NS * RPT

EDGE_BLOCK = 512

_SC_PARAMS = pltpu.CompilerParams(needs_layout_passes=False,
                                  use_tc_tiling_on_sc=False)


def _silu(x):
    return x * (1.0 / (1.0 + jnp.exp(-x)))


def _prep_body(h_ref, w_ref, b_ref, p_ref, q_ref):
    h = h_ref[...]
    p_ref[...] = (jnp.dot(h, w_ref[0:D, :], preferred_element_type=jnp.float32)
                  + b_ref[...]).astype(jnp.bfloat16)
    q_ref[...] = jnp.dot(h, w_ref[D:2 * D, :],
                         preferred_element_type=jnp.float32).astype(jnp.bfloat16)


def _prep(h, W_e1, b_e1):
    return pl.pallas_call(
        _prep_body,
        out_shape=[
            jax.ShapeDtypeStruct((N, H), jnp.bfloat16),
            jax.ShapeDtypeStruct((N, H), jnp.bfloat16),
        ],
    )(h, W_e1, b_e1.reshape(1, H))


def _sc_gather(P, Q, coordflat, w_r, row, col):
    mesh = plsc.VectorSubcoreMesh(core_axis_name="core",
                                  subcore_axis_name="subcore")

    nbuf = 2
    scratch = [
        pltpu.VMEM((4 * N,), jnp.float32),
        pltpu.VMEM((H,), jnp.bfloat16),
    ]
    for _ in range(nbuf):
        scratch += [
            pltpu.VMEM((GW,), jnp.int32),
            pltpu.VMEM((GW,), jnp.int32),
            pltpu.VMEM((GW, H), jnp.bfloat16),
            pltpu.VMEM((GW, H), jnp.bfloat16),
            pltpu.VMEM((GW,), jnp.float32),
            pltpu.VMEM((GW,), jnp.float32),
            pltpu.VMEM((GW,), jnp.float32),
            pltpu.VMEM((GW,), jnp.float32),
            pltpu.SemaphoreType.DMA,
            pltpu.SemaphoreType.DMA,
            pltpu.SemaphoreType.DMA,
            pltpu.SemaphoreType.DMA,
        ]

    @functools.partial(
        pl.kernel,
        mesh=mesh,
        compiler_params=_SC_PARAMS,
        out_type=[
            jax.ShapeDtypeStruct((E, H), jnp.bfloat16),
            jax.ShapeDtypeStruct((E,), jnp.float32),
            jax.ShapeDtypeStruct((E,), jnp.float32),
            jax.ShapeDtypeStruct((E,), jnp.float32),
        ],
        scratch_types=scratch,
    )
    def k(p_hbm, q_hbm, c4_hbm, wr_hbm, row_hbm, col_hbm,
          s_hbm, dx_hbm, dy_hbm, dz_hbm, coordv, wrv, *bufs):
        banks = [bufs[i * 12:(i + 1) * 12] for i in range(nbuf)]
        wid = lax.axis_index("subcore") * NC + lax.axis_index("core")
        pltpu.sync_copy(c4_hbm, coordv)
        pltpu.sync_copy(wr_hbm, wrv)
        wr_regs = [wrv[pl.ds(c * 32, 32)] for c in range(H // 32)]

        def chunk_base(j):
            return pl.multiple_of(wid * EPT + j * GW, GW)

        def issue_idx(j, b):
            (idxr, idxc, _, _, _, _, _, _, sem_i, _, _, _) = banks[b]
            base = chunk_base(j)
            pltpu.async_copy(row_hbm.at[pl.ds(base, GW)], idxr, sem_i)
            pltpu.async_copy(col_hbm.at[pl.ds(base, GW)], idxc, sem_i)

        def wait_idx(b):
            (idxr, idxc, _, _, _, _, _, _, sem_i, _, _, _) = banks[b]
            pltpu.make_async_copy(row_hbm.at[pl.ds(0, GW)], idxr, sem_i).wait()
            pltpu.make_async_copy(col_hbm.at[pl.ds(0, GW)], idxc, sem_i).wait()

        def issue_gather(b):
            (idxr, idxc, sbuf, qbuf, _, _, _, _, _, sem_p, sem_q, _) = banks[b]
            pltpu.async_copy(p_hbm.at[idxr], sbuf, sem_p)
            pltpu.async_copy(q_hbm.at[idxc], qbuf, sem_q)

        def wait_stores(b):
            (_, _, sbuf, _, dxv, dyv, dzv, _, _, _, _, sem_s) = banks[b]
            base0 = pl.ds(0, GW)
            pltpu.make_async_copy(sbuf, s_hbm.at[base0], sem_s).wait()
            pltpu.make_async_copy(dxv, dx_hbm.at[base0], sem_s).wait()
            pltpu.make_async_copy(dyv, dy_hbm.at[base0], sem_s).wait()
            pltpu.make_async_copy(dzv, dz_hbm.at[base0], sem_s).wait()

        def process(j, b):
            (idxr, idxc, sbuf, qbuf, dxv, dyv, dzv, radv,
             _, sem_p, sem_q, sem_s) = banks[b]
            base = chunk_base(j)
            for g in range(GW // 16):
                sl = pl.ds(g * 16, 16)
                ir4 = idxr[sl] * 4
                ic4 = idxc[sl] * 4
                d = []
                for c in range(3):
                    dc = (plsc.load_gather(coordv, [ir4 + c])
                          - plsc.load_gather(coordv, [ic4 + c]))
                    d.append(dc)
                dxv[sl] = d[0]
                dyv[sl] = d[1]
                dzv[sl] = d[2]
                radv[sl] = d[0] * d[0] + d[1] * d[1] + d[2] * d[2]
            pltpu.make_async_copy(p_hbm.at[idxr], sbuf, sem_p).wait()
            pltpu.make_async_copy(q_hbm.at[idxc], qbuf, sem_q).wait()

            @pl.loop(0, GW // 16)
            def _(g):
                g16 = pl.multiple_of(g * 16, 16)
                rad16 = radv[pl.ds(g16, 16)]
                for e in range(16):
                    i = g16 + e
                    rs16 = jnp.full((16,), rad16[e], jnp.float32)
                    rsv = plsc.pack(rs16, rs16,
                                    format=plsc.PackFormat.INTERLEAVED)
                    for c in range(H // 32):
                        sl = pl.ds(c * 32, 32)
                        sbuf[i, sl] = sbuf[i, sl] + qbuf[i, sl] + rsv * wr_regs[c]

            pltpu.async_copy(sbuf, s_hbm.at[pl.ds(base, GW)], sem_s)
            pltpu.async_copy(dxv, dx_hbm.at[pl.ds(base, GW)], sem_s)
            pltpu.async_copy(dyv, dy_hbm.at[pl.ds(base, GW)], sem_s)
            pltpu.async_copy(dzv, dz_hbm.at[pl.ds(base, GW)], sem_s)

        issue_idx(0, 0)
        issue_idx(1, 1)
        wait_idx(0)
        issue_gather(0)

        def step(j, b, has_next, next2):
            bn = 1 - b
            if has_next:
                wait_idx(bn)

                @pl.when(j >= 1)
                def _():
                    wait_stores(bn)

                issue_gather(bn)
            process(j, b)
            if next2 is True:
                issue_idx(j + 2, b)
            elif next2 is not None:
                @pl.when(next2)
                def _():
                    issue_idx(j + 2, b)

        @pl.loop(0, (CHUNKS - 1) // 2)
        def _(i):
            j = i * 2
            step(j, 0, True, True)
            step(j + 1, 1, True, j + 3 < CHUNKS)

        step(CHUNKS - 1, (CHUNKS - 1) % 2, False, None)
        wait_stores(0)
        wait_stores(1)

    return k(P, Q, coordflat, w_r, row, col)


def _edge_mlp_body(s_ref, ea_ref, dx_ref, dy_ref, dz_ref,
                   wea_ref, we2_ref, be2_ref, wc1_ref, bc1_ref, wc2_ref,
                   ef_ref, tx_ref, ty_ref, tz_ref):
    bf = jnp.bfloat16
    s = s_ref[...]
    m = _silu(s + jnp.dot(ea_ref[...].astype(bf), wea_ref[...],
                          preferred_element_type=jnp.float32))
    ef = _silu(jnp.dot(m.astype(bf), we2_ref[...],
                       preferred_element_type=jnp.float32) + be2_ref[...])
    p = _silu(jnp.dot(ef.astype(bf), wc1_ref[...],
                      preferred_element_type=jnp.float32) + bc1_ref[...])
    cm = lax.dot_general(wc2_ref[...], p.astype(bf), (((1,), (1,)), ((), ())),
                         preferred_element_type=jnp.float32)
    cm1 = cm.reshape((cm.shape[1],))
    ef_ref[...] = ef
    tx_ref[...] = dx_ref[...] * cm1
    ty_ref[...] = dy_ref[...] * cm1
    tz_ref[...] = dz_ref[...] * cm1


def _edge_mlp(S, dx, dy, dz, edge_attr, W_e1, b_e2, W_e2, W_c1, b_c1, W_c2):
    B = EDGE_BLOCK
    grid = (E // B,)
    full = lambda shape: pl.BlockSpec(shape, lambda i: (0, 0))
    vec = pl.BlockSpec((B,), lambda i: (i,))
    W_ea = W_e1[2 * D + 1:2 * D + 1 + DE, :]
    return pl.pallas_call(
        _edge_mlp_body,
        grid=grid,
        in_specs=[
            pl.BlockSpec((B, H), lambda i: (i, 0)),
            pl.BlockSpec((B, DE), lambda i: (i, 0)),
            vec, vec, vec,
            full((DE, H)),
            full((H, H)),
            full((1, H)),
            full((H, H)),
            full((1, H)),
            full((1, H)),
        ],
        out_specs=[
            pl.BlockSpec((B, H), lambda i: (i, 0)),
            vec, vec, vec,
        ],
        out_shape=[
            jax.ShapeDtypeStruct((E, H), jnp.float32),
            jax.ShapeDtypeStruct((E,), jnp.float32),
            jax.ShapeDtypeStruct((E,), jnp.float32),
            jax.ShapeDtypeStruct((E,), jnp.float32),
        ],
    )(S, edge_attr, dx, dy, dz, W_ea.astype(jnp.bfloat16),
      W_e2.astype(jnp.bfloat16), b_e2.reshape(1, H),
      W_c1.astype(jnp.bfloat16), b_c1.reshape(1, H),
      W_c2.reshape(1, H).astype(jnp.bfloat16))


def _sc_scatter(row, EF, tx, ty, tz, zH):
    mesh = plsc.VectorSubcoreMesh(core_axis_name="core",
                                  subcore_axis_name="subcore")
    scalar_mesh = plsc.ScalarSubcoreMesh(axis_name="core", num_cores=NC)

    def tec_fn(row_hbm, ef_hbm, tx_hbm, ty_hbm, tz_hbm, zh_hbm,
               aggh_hbm, aggt_hbm, accH, accT):
        core = lax.axis_index("core")
        sid = lax.axis_index("subcore")
        wid = sid * NC + core
        rows0 = sid * RPT

        nbuf = 3

        def body(*bufs):
            _tec_body(row_hbm, ef_hbm, tx_hbm, ty_hbm, tz_hbm, zh_hbm,
                      aggh_hbm, aggt_hbm, accH, accT, bufs,
                      core, sid, wid, rows0)

        scoped = []
        for _ in range(nbuf):
            scoped += [
                pltpu.VMEM((GW,), jnp.int32),
                pltpu.VMEM((GW, H), jnp.float32),
                pltpu.VMEM((GW,), jnp.float32),
                pltpu.VMEM((GW,), jnp.float32),
                pltpu.VMEM((GW,), jnp.float32),
                pltpu.VMEM((GW, 16), jnp.float32),
                pltpu.SemaphoreType.DMA,
            ]
        pl.run_scoped(body, *scoped)

    def _tec_body(row_hbm, ef_hbm, tx_hbm, ty_hbm, tz_hbm, zh_hbm,
                  aggh_hbm, aggt_hbm, accH, accT, bufs,
                  core, sid, wid, rows0):
        nbuf = 3
        banks = [bufs[i * 7:(i + 1) * 7] for i in range(nbuf)]
        pltpu.sync_copy(zh_hbm, accH.at[pl.ds(rows0, RPT)])

        zero16 = jnp.zeros((16,), jnp.float32)
        one16 = jnp.ones((16,), jnp.float32)
        for b in range(nbuf):
            t4v = banks[b][5]
            for g in range(GW // 16):
                loc = g * 16 + lax.iota(jnp.int32, 16)
                for c in range(16):
                    plsc.store_scatter(
                        t4v, [loc, jnp.full((16,), c, jnp.int32)], zero16)

        z16 = banks[0][5]

        @pl.loop(0, 7)
        def _(jj):
            pltpu.sync_copy(z16, accT.at[pl.ds(rows0 + jj * GW, GW)])

        pltpu.sync_copy(z16.at[pl.ds(0, 72)],
                        accT.at[pl.ds(rows0 + 7 * GW, 72)])

        for b in range(nbuf):
            t4v = banks[b][5]
            for g in range(GW // 16):
                loc = g * 16 + lax.iota(jnp.int32, 16)
                plsc.store_scatter(t4v, [loc, jnp.full((16,), 3, jnp.int32)],
                                   one16)

        plsc.subcore_barrier()

        def issue_loads(j, b):
            (idxv, efv, txv, tyv, tzv, _, sem) = banks[b]
            base = pl.multiple_of(wid * EPT + j * GW, GW)
            pltpu.async_copy(row_hbm.at[pl.ds(base, GW)], idxv, sem)
            pltpu.async_copy(ef_hbm.at[pl.ds(base, GW)], efv, sem)
            pltpu.async_copy(tx_hbm.at[pl.ds(base, GW)], txv, sem)
            pltpu.async_copy(ty_hbm.at[pl.ds(base, GW)], tyv, sem)
            pltpu.async_copy(tz_hbm.at[pl.ds(base, GW)], tzv, sem)

        def wait_loads(b):
            (idxv, efv, txv, tyv, tzv, _, sem) = banks[b]
            z = pl.ds(0, GW)
            pltpu.make_async_copy(row_hbm.at[z], idxv, sem).wait()
            pltpu.make_async_copy(ef_hbm.at[z], efv, sem).wait()
            pltpu.make_async_copy(tx_hbm.at[z], txv, sem).wait()
            pltpu.make_async_copy(ty_hbm.at[z], tyv, sem).wait()
            pltpu.make_async_copy(tz_hbm.at[z], tzv, sem).wait()

        def step(j, b, next2):
            (idxv, efv, txv, tyv, tzv, t4v, _) = banks[b]
            wait_loads(b)
            if next2 is True:
                issue_loads(j + 2, (b + 2) % nbuf)
            elif next2 is not None:
                @pl.when(next2)
                def _():
                    issue_loads(j + 2, (b + 2) % nbuf)
            for g in range(GW // 16):
                sl = pl.ds(g * 16, 16)
                loc = g * 16 + lax.iota(jnp.int32, 16)
                plsc.store_scatter(t4v, [loc, jnp.full((16,), 0, jnp.int32)],
                                   txv[sl])
                plsc.store_scatter(t4v, [loc, jnp.full((16,), 1, jnp.int32)],
                                   tyv[sl])
                plsc.store_scatter(t4v, [loc, jnp.full((16,), 2, jnp.int32)],
                                   tzv[sl])
            pltpu.sync_copy(efv, accH.at[idxv], add=True)
            pltpu.sync_copy(t4v, accT.at[idxv], add=True)

        issue_loads(0, 0)
        issue_loads(1, 1)

        @pl.loop(0, CHUNKS // nbuf)
        def _(i):
            j = i * nbuf
            step(j, 0, True)
            step(j + 1, 1, True)
            step(j + 2, 2, j + 4 < CHUNKS)

        step(CHUNKS - 2, 0, None)
        step(CHUNKS - 1, 1, None)

        plsc.subcore_barrier()
        pltpu.sync_copy(accH.at[pl.ds(rows0, RPT)],
                        aggh_hbm.at[core, pl.ds(rows0, RPT)])
        pltpu.sync_copy(accT.at[pl.ds(rows0, RPT)],
                        aggt_hbm.at[core, pl.ds(rows0, RPT)])

    def scs_fn(row_hbm, ef_hbm, tx_hbm, ty_hbm, tz_hbm, zh_hbm,
               aggh_hbm, aggt_hbm, accH, accT):
        pass

    k = pl.kernel(
        [tec_fn, scs_fn],
        out_type=[
            jax.ShapeDtypeStruct((NC, NP, H), jnp.float32),
            jax.ShapeDtypeStruct((NC, NP, 16), jnp.float32),
        ],
        mesh=[mesh, scalar_mesh],
        compiler_params=_SC_PARAMS,
        scratch_types=[
            pltpu.VMEM_SHARED((NP, H), jnp.float32),
            pltpu.VMEM_SHARED((NP, 16), jnp.float32),
        ],
    )
    return k(row, EF, tx, ty, tz, zH)


def _node_body(h_ref, c4_ref, aggh_ref, aggt_ref, wn1_ref, bn1_ref, wn2_ref,
               bn2_ref, hout_ref, cout_ref):
    h = h_ref[...]
    agg = aggh_ref[0, 0:N, :] + aggh_ref[1, 0:N, :]
    u = _silu(jnp.dot(h, wn1_ref[0:D, :], preferred_element_type=jnp.float32)
              + jnp.dot(agg, wn1_ref[D:D + H, :],
                        preferred_element_type=jnp.float32)
              + bn1_ref[...])
    hout_ref[...] = h + jnp.dot(u, wn2_ref[...],
                                preferred_element_type=jnp.float32) + bn2_ref[...]
    aggt = aggt_ref[0, 0:N, :] + aggt_ref[1, 0:N, :]
    cnt = jnp.maximum(aggt[:, 3:4], 1.0)
    cout_ref[...] = c4_ref[...] + aggt[:, 0:4] / cnt


def _node_update(h, coord4, agg_h, agg_t, W_n1, b_n1, W_n2, b_n2):
    return pl.pallas_call(
        _node_body,
        out_shape=[
            jax.ShapeDtypeStruct((N, D), jnp.float32),
            jax.ShapeDtypeStruct((N, 4), jnp.float32),
        ],
    )(h, coord4, agg_h, agg_t, W_n1, b_n1.reshape(1, H), W_n2,
      b_n2.reshape(1, D))


def kernel(h, edge_index, coord, edge_attr, W_e1, b_e1, W_e2, b_e2,
           W_n1, b_n1, W_n2, b_n2, W_c1, b_c1, W_c2):
    row = edge_index[0]
    col = edge_index[1]
    coord4 = jnp.concatenate([coord, jnp.zeros((N, 1), jnp.float32)], axis=1)
    coordflat = coord4.reshape(4 * N)
    w_r = W_e1[2 * D].astype(jnp.bfloat16)

    P, Q = _prep(h, W_e1, b_e1)
    S, dx, dy, dz = _sc_gather(P, Q, coordflat, w_r, row, col)
    EF, tx, ty, tz = _edge_mlp(S, dx, dy, dz, edge_attr, W_e1, b_e2, W_e2,
                               W_c1, b_c1, W_c2)
    zH = jnp.zeros((RPT, H), jnp.float32)
    aggH, aggT = _sc_scatter(row, EF, tx, ty, tz, zH)
    h_out, c_out4 = _node_update(h, coord4, aggH, aggT, W_n1, b_n1,
                                 W_n2, b_n2)
    return (h_out, c_out4[:, 0:3], edge_attr)


# speedup vs baseline: 1.2428x; 1.2428x over previous
import functools

import jax
import jax.numpy as jnp
from jax import lax
from jax.experimental import pallas as pl
from jax.experimental.pallas import tpu as pltpu
from jax.experimental.pallas import tpu_sc as plsc

N = 10000
E = 320000
D = 128
H = 128
DE = 16

NC = 2
NS = 16
NW = NC * NS
GW = 80
SEG = 5
ES = E // SEG
EPT = ES // NW
CHUNKS = EPT // GW
RPT = 632
NP = NS * RPT

EDGE_BLOCK = 512

_SC_PARAMS = pltpu.CompilerParams(needs_layout_passes=False,
                                  use_tc_tiling_on_sc=False)


def _silu(x):
    return x * (1.0 / (1.0 + jnp.exp(-x)))


def _prep_body(h_ref, w_ref, b_ref, p_ref, q_ref):
    h = h_ref[...]
    p_ref[...] = (jnp.dot(h, w_ref[0:D, :], preferred_element_type=jnp.float32)
                  + b_ref[...])
    q_ref[...] = jnp.dot(h, w_ref[D:2 * D, :], preferred_element_type=jnp.float32)


def _prep(h, W_e1, b_e1):
    return pl.pallas_call(
        _prep_body,
        out_shape=[
            jax.ShapeDtypeStruct((N, H), jnp.float32),
            jax.ShapeDtypeStruct((N, H), jnp.float32),
        ],
    )(h, W_e1, b_e1.reshape(1, H))


def _sc_gather(P, Q, coordflat, w_r, row, col):
    mesh = plsc.VectorSubcoreMesh(core_axis_name="core",
                                  subcore_axis_name="subcore")

    nbuf = 2
    scratch = [
        pltpu.VMEM((4 * N,), jnp.float32),
        pltpu.VMEM((H,), jnp.float32),
    ]
    for _ in range(nbuf):
        scratch += [
            pltpu.VMEM((GW,), jnp.int32),
            pltpu.VMEM((GW,), jnp.int32),
            pltpu.VMEM((GW, H), jnp.float32),
            pltpu.VMEM((GW, H), jnp.float32),
            pltpu.VMEM((GW,), jnp.float32),
            pltpu.VMEM((GW,), jnp.float32),
            pltpu.VMEM((GW,), jnp.float32),
            pltpu.VMEM((GW,), jnp.float32),
            pltpu.SemaphoreType.DMA,
            pltpu.SemaphoreType.DMA,
            pltpu.SemaphoreType.DMA,
            pltpu.SemaphoreType.DMA,
        ]

    @functools.partial(
        pl.kernel,
        mesh=mesh,
        compiler_params=_SC_PARAMS,
        out_type=[
            jax.ShapeDtypeStruct((ES, H), jnp.float32),
            jax.ShapeDtypeStruct((ES,), jnp.float32),
            jax.ShapeDtypeStruct((ES,), jnp.float32),
            jax.ShapeDtypeStruct((ES,), jnp.float32),
        ],
        scratch_types=scratch,
    )
    def k(p_hbm, q_hbm, c4_hbm, wr_hbm, row_hbm, col_hbm,
          s_hbm, dx_hbm, dy_hbm, dz_hbm, coordv, wrv, *bufs):
        banks = [bufs[i * 12:(i + 1) * 12] for i in range(nbuf)]
        wid = lax.axis_index("subcore") * NC + lax.axis_index("core")
        pltpu.sync_copy(c4_hbm, coordv)
        pltpu.sync_copy(wr_hbm, wrv)
        wr_regs = [wrv[pl.ds(c * 16, 16)] for c in range(H // 16)]

        def chunk_base(j):
            return pl.multiple_of(wid * EPT + j * GW, GW)

        def issue_idx(j, b):
            (idxr, idxc, _, _, _, _, _, _, sem_i, _, _, _) = banks[b]
            base = chunk_base(j)
            pltpu.async_copy(row_hbm.at[pl.ds(base, GW)], idxr, sem_i)
            pltpu.async_copy(col_hbm.at[pl.ds(base, GW)], idxc, sem_i)

        def wait_idx(b):
            (idxr, idxc, _, _, _, _, _, _, sem_i, _, _, _) = banks[b]
            pltpu.make_async_copy(row_hbm.at[pl.ds(0, GW)], idxr, sem_i).wait()
            pltpu.make_async_copy(col_hbm.at[pl.ds(0, GW)], idxc, sem_i).wait()

        def issue_gather(b):
            (idxr, idxc, sbuf, qbuf, _, _, _, _, _, sem_p, sem_q, _) = banks[b]
            pltpu.async_copy(p_hbm.at[idxr], sbuf, sem_p)
            pltpu.async_copy(q_hbm.at[idxc], qbuf, sem_q)

        def wait_stores(b):
            (_, _, sbuf, _, dxv, dyv, dzv, _, _, _, _, sem_s) = banks[b]
            base0 = pl.ds(0, GW)
            pltpu.make_async_copy(sbuf, s_hbm.at[base0], sem_s).wait()
            pltpu.make_async_copy(dxv, dx_hbm.at[base0], sem_s).wait()
            pltpu.make_async_copy(dyv, dy_hbm.at[base0], sem_s).wait()
            pltpu.make_async_copy(dzv, dz_hbm.at[base0], sem_s).wait()

        def process(j, b):
            (idxr, idxc, sbuf, qbuf, dxv, dyv, dzv, radv,
             _, sem_p, sem_q, sem_s) = banks[b]
            base = chunk_base(j)
            for g in range(GW // 16):
                sl = pl.ds(g * 16, 16)
                ir4 = idxr[sl] * 4
                ic4 = idxc[sl] * 4
                d = []
                for c in range(3):
                    dc = (plsc.load_gather(coordv, [ir4 + c])
                          - plsc.load_gather(coordv, [ic4 + c]))
                    d.append(dc)
                dxv[sl] = d[0]
                dyv[sl] = d[1]
                dzv[sl] = d[2]
                radv[sl] = d[0] * d[0] + d[1] * d[1] + d[2] * d[2]
            pltpu.make_async_copy(p_hbm.at[idxr], sbuf, sem_p).wait()
            pltpu.make_async_copy(q_hbm.at[idxc], qbuf, sem_q).wait()

            @pl.loop(0, GW // 16)
            def _(g):
                g16 = pl.multiple_of(g * 16, 16)
                rad16 = radv[pl.ds(g16, 16)]
                for e in range(16):
                    i = g16 + e
                    rs = rad16[e]
                    for c in range(H // 16):
                        sl = pl.ds(c * 16, 16)
                        sbuf[i, sl] = sbuf[i, sl] + qbuf[i, sl] + rs * wr_regs[c]

            pltpu.async_copy(sbuf, s_hbm.at[pl.ds(base, GW)], sem_s)
            pltpu.async_copy(dxv, dx_hbm.at[pl.ds(base, GW)], sem_s)
            pltpu.async_copy(dyv, dy_hbm.at[pl.ds(base, GW)], sem_s)
            pltpu.async_copy(dzv, dz_hbm.at[pl.ds(base, GW)], sem_s)

        issue_idx(0, 0)
        issue_idx(1, 1)
        wait_idx(0)
        issue_gather(0)

        def step(j, b, has_next, next2):
            bn = 1 - b
            if has_next:
                wait_idx(bn)

                @pl.when(j >= 1)
                def _():
                    wait_stores(bn)

                issue_gather(bn)
            process(j, b)
            if next2 is True:
                issue_idx(j + 2, b)
            elif next2 is not None:
                @pl.when(next2)
                def _():
                    issue_idx(j + 2, b)

        @pl.loop(0, (CHUNKS - 1) // 2)
        def _(i):
            j = i * 2
            step(j, 0, True, True)
            step(j + 1, 1, True, j + 3 < CHUNKS)

        step(CHUNKS - 1, (CHUNKS - 1) % 2, False, None)
        wait_stores(0)
        wait_stores(1)

    return k(P, Q, coordflat, w_r, row, col)


def _edge_mlp_body(s_ref, ea_ref, dx_ref, dy_ref, dz_ref,
                   wea_ref, we2_ref, be2_ref, wc1_ref, bc1_ref, wc2_ref,
                   ef_ref, tx_ref, ty_ref, tz_ref):
    bf = jnp.bfloat16
    s = s_ref[...]
    m = _silu(s + jnp.dot(ea_ref[...].astype(bf), wea_ref[...],
                          preferred_element_type=jnp.float32))
    ef = _silu(jnp.dot(m.astype(bf), we2_ref[...],
                       preferred_element_type=jnp.float32) + be2_ref[...])
    p = _silu(jnp.dot(ef.astype(bf), wc1_ref[...],
                      preferred_element_type=jnp.float32) + bc1_ref[...])
    cm = lax.dot_general(wc2_ref[...], p.astype(bf), (((1,), (1,)), ((), ())),
                         preferred_element_type=jnp.float32)
    cm1 = cm.reshape((cm.shape[1],))
    ef_ref[...] = ef
    tx_ref[...] = dx_ref[...] * cm1
    ty_ref[...] = dy_ref[...] * cm1
    tz_ref[...] = dz_ref[...] * cm1


def _edge_mlp(S, dx, dy, dz, edge_attr, W_e1, b_e2, W_e2, W_c1, b_c1, W_c2):
    B = EDGE_BLOCK
    grid = (ES // B,)
    full = lambda shape: pl.BlockSpec(shape, lambda i: (0, 0))
    vec = pl.BlockSpec((B,), lambda i: (i,))
    W_ea = W_e1[2 * D + 1:2 * D + 1 + DE, :]
    return pl.pallas_call(
        _edge_mlp_body,
        grid=grid,
        in_specs=[
            pl.BlockSpec((B, H), lambda i: (i, 0)),
            pl.BlockSpec((B, DE), lambda i: (i, 0)),
            vec, vec, vec,
            full((DE, H)),
            full((H, H)),
            full((1, H)),
            full((H, H)),
            full((1, H)),
            full((1, H)),
        ],
        out_specs=[
            pl.BlockSpec((B, H), lambda i: (i, 0)),
            vec, vec, vec,
        ],
        out_shape=[
            jax.ShapeDtypeStruct((ES, H), jnp.float32),
            jax.ShapeDtypeStruct((ES,), jnp.float32),
            jax.ShapeDtypeStruct((ES,), jnp.float32),
            jax.ShapeDtypeStruct((ES,), jnp.float32),
        ],
    )(S, edge_attr, dx, dy, dz, W_ea.astype(jnp.bfloat16),
      W_e2.astype(jnp.bfloat16), b_e2.reshape(1, H),
      W_c1.astype(jnp.bfloat16), b_c1.reshape(1, H),
      W_c2.reshape(1, H).astype(jnp.bfloat16))


def _sc_scatter(row, EF, tx, ty, tz, zH):
    mesh = plsc.VectorSubcoreMesh(core_axis_name="core",
                                  subcore_axis_name="subcore")
    scalar_mesh = plsc.ScalarSubcoreMesh(axis_name="core", num_cores=NC)

    def tec_fn(row_hbm, ef_hbm, tx_hbm, ty_hbm, tz_hbm, zh_hbm,
               aggh_hbm, aggt_hbm, accH, accT):
        core = lax.axis_index("core")
        sid = lax.axis_index("subcore")
        wid = sid * NC + core
        rows0 = sid * RPT

        nbuf = 3

        def body(*bufs):
            _tec_body(row_hbm, ef_hbm, tx_hbm, ty_hbm, tz_hbm, zh_hbm,
                      aggh_hbm, aggt_hbm, accH, accT, bufs,
                      core, sid, wid, rows0)

        scoped = []
        for _ in range(nbuf):
            scoped += [
                pltpu.VMEM((GW,), jnp.int32),
                pltpu.VMEM((GW, H), jnp.float32),
                pltpu.VMEM((GW,), jnp.float32),
                pltpu.VMEM((GW,), jnp.float32),
                pltpu.VMEM((GW,), jnp.float32),
                pltpu.VMEM((GW, 16), jnp.float32),
                pltpu.SemaphoreType.DMA,
            ]
        pl.run_scoped(body, *scoped)

    def _tec_body(row_hbm, ef_hbm, tx_hbm, ty_hbm, tz_hbm, zh_hbm,
                  aggh_hbm, aggt_hbm, accH, accT, bufs,
                  core, sid, wid, rows0):
        nbuf = 3
        banks = [bufs[i * 7:(i + 1) * 7] for i in range(nbuf)]
        pltpu.sync_copy(zh_hbm, accH.at[pl.ds(rows0, RPT)])

        zero16 = jnp.zeros((16,), jnp.float32)
        one16 = jnp.ones((16,), jnp.float32)
        for b in range(nbuf):
            t4v = banks[b][5]
            for g in range(GW // 16):
                loc = g * 16 + lax.iota(jnp.int32, 16)
                for c in range(16):
                    plsc.store_scatter(
                        t4v, [loc, jnp.full((16,), c, jnp.int32)], zero16)

        z16 = banks[0][5]

        @pl.loop(0, 7)
        def _(jj):
            pltpu.sync_copy(z16, accT.at[pl.ds(rows0 + jj * GW, GW)])

        pltpu.sync_copy(z16.at[pl.ds(0, 72)],
                        accT.at[pl.ds(rows0 + 7 * GW, 72)])

        for b in range(nbuf):
            t4v = banks[b][5]
            for g in range(GW // 16):
                loc = g * 16 + lax.iota(jnp.int32, 16)
                plsc.store_scatter(t4v, [loc, jnp.full((16,), 3, jnp.int32)],
                                   one16)

        plsc.subcore_barrier()

        def issue_loads(j, b):
            (idxv, efv, txv, tyv, tzv, _, sem) = banks[b]
            base = pl.multiple_of(wid * EPT + j * GW, GW)
            pltpu.async_copy(row_hbm.at[pl.ds(base, GW)], idxv, sem)
            pltpu.async_copy(ef_hbm.at[pl.ds(base, GW)], efv, sem)
            pltpu.async_copy(tx_hbm.at[pl.ds(base, GW)], txv, sem)
            pltpu.async_copy(ty_hbm.at[pl.ds(base, GW)], tyv, sem)
            pltpu.async_copy(tz_hbm.at[pl.ds(base, GW)], tzv, sem)

        def wait_loads(b):
            (idxv, efv, txv, tyv, tzv, _, sem) = banks[b]
            z = pl.ds(0, GW)
            pltpu.make_async_copy(row_hbm.at[z], idxv, sem).wait()
            pltpu.make_async_copy(ef_hbm.at[z], efv, sem).wait()
            pltpu.make_async_copy(tx_hbm.at[z], txv, sem).wait()
            pltpu.make_async_copy(ty_hbm.at[z], tyv, sem).wait()
            pltpu.make_async_copy(tz_hbm.at[z], tzv, sem).wait()

        def step(j, b, next2):
            (idxv, efv, txv, tyv, tzv, t4v, _) = banks[b]
            wait_loads(b)
            if next2 is True:
                issue_loads(j + 2, (b + 2) % nbuf)
            elif next2 is not None:
                @pl.when(next2)
                def _():
                    issue_loads(j + 2, (b + 2) % nbuf)
            for g in range(GW // 16):
                sl = pl.ds(g * 16, 16)
                loc = g * 16 + lax.iota(jnp.int32, 16)
                plsc.store_scatter(t4v, [loc, jnp.full((16,), 0, jnp.int32)],
                                   txv[sl])
                plsc.store_scatter(t4v, [loc, jnp.full((16,), 1, jnp.int32)],
                                   tyv[sl])
                plsc.store_scatter(t4v, [loc, jnp.full((16,), 2, jnp.int32)],
                                   tzv[sl])
            pltpu.sync_copy(efv, accH.at[idxv], add=True)
            pltpu.sync_copy(t4v, accT.at[idxv], add=True)

        issue_loads(0, 0)
        issue_loads(1, 1)
        m3 = CHUNKS // nbuf

        @pl.loop(0, m3)
        def _(i):
            j = i * nbuf
            step(j, 0, True)
            step(j + 1, 1, True)
            step(j + 2, 2, j + 4 < CHUNKS)

        for t in range(CHUNKS - nbuf * m3):
            step(nbuf * m3 + t, t, None)

        plsc.subcore_barrier()
        pltpu.sync_copy(accH.at[pl.ds(rows0, RPT)],
                        aggh_hbm.at[core, pl.ds(rows0, RPT)])
        pltpu.sync_copy(accT.at[pl.ds(rows0, RPT)],
                        aggt_hbm.at[core, pl.ds(rows0, RPT)])

    def scs_fn(row_hbm, ef_hbm, tx_hbm, ty_hbm, tz_hbm, zh_hbm,
               aggh_hbm, aggt_hbm, accH, accT):
        pass

    k = pl.kernel(
        [tec_fn, scs_fn],
        out_type=[
            jax.ShapeDtypeStruct((NC, NP, H), jnp.float32),
            jax.ShapeDtypeStruct((NC, NP, 16), jnp.float32),
        ],
        mesh=[mesh, scalar_mesh],
        compiler_params=_SC_PARAMS,
        scratch_types=[
            pltpu.VMEM_SHARED((NP, H), jnp.float32),
            pltpu.VMEM_SHARED((NP, 16), jnp.float32),
        ],
    )
    return k(row, EF, tx, ty, tz, zH)


NODE_BLOCK = 1000


def _node_body(h_ref, c4_ref, aggh_ref, aggt_ref, wn1_ref, bn1_ref, wn2_ref,
               bn2_ref, hout_ref, cout_ref):
    h = h_ref[...]
    agg = aggh_ref[0]
    aggt = aggt_ref[0]
    for k in range(1, 2 * SEG):
        agg = agg + aggh_ref[k]
        aggt = aggt + aggt_ref[k]
    u = _silu(jnp.dot(h, wn1_ref[0:D, :], preferred_element_type=jnp.float32)
              + jnp.dot(agg, wn1_ref[D:D + H, :],
                        preferred_element_type=jnp.float32)
              + bn1_ref[...])
    hout_ref[...] = h + jnp.dot(u, wn2_ref[...],
                                preferred_element_type=jnp.float32) + bn2_ref[...]
    cnt = jnp.maximum(aggt[:, 3:4], 1.0)
    cout_ref[...] = c4_ref[...] + aggt[:, 0:4] / cnt


def _node_update(h, coord4, agg_h, agg_t, W_n1, b_n1, W_n2, b_n2):
    B = NODE_BLOCK
    full = lambda shape: pl.BlockSpec(shape, lambda i: (0, 0))
    return pl.pallas_call(
        _node_body,
        grid=(N // B,),
        in_specs=[
            pl.BlockSpec((B, D), lambda i: (i, 0)),
            pl.BlockSpec((B, 4), lambda i: (i, 0)),
            pl.BlockSpec((2 * SEG, B, H), lambda i: (0, i, 0)),
            pl.BlockSpec((2 * SEG, B, 16), lambda i: (0, i, 0)),
            full((2 * D, H)),
            full((1, H)),
            full((H, D)),
            full((1, D)),
        ],
        out_specs=[
            pl.BlockSpec((B, D), lambda i: (i, 0)),
            pl.BlockSpec((B, 4), lambda i: (i, 0)),
        ],
        out_shape=[
            jax.ShapeDtypeStruct((N, D), jnp.float32),
            jax.ShapeDtypeStruct((N, 4), jnp.float32),
        ],
    )(h, coord4, agg_h, agg_t, W_n1, b_n1.reshape(1, H), W_n2,
      b_n2.reshape(1, D))


def kernel(h, edge_index, coord, edge_attr, W_e1, b_e1, W_e2, b_e2,
           W_n1, b_n1, W_n2, b_n2, W_c1, b_c1, W_c2):
    row = edge_index[0]
    col = edge_index[1]
    coord4 = jnp.concatenate([coord, jnp.zeros((N, 1), jnp.float32)], axis=1)
    coordflat = coord4.reshape(4 * N)
    w_r = W_e1[2 * D]

    P, Q = _prep(h, W_e1, b_e1)
    zH = jnp.zeros((RPT, H), jnp.float32)
    aggHs, aggTs = [], []
    for k in range(SEG):
        sl = slice(k * ES, (k + 1) * ES)
        row_k, col_k = row[sl], col[sl]
        S, dx, dy, dz = _sc_gather(P, Q, coordflat, w_r, row_k, col_k)
        EF, tx, ty, tz = _edge_mlp(S, dx, dy, dz, edge_attr[sl], W_e1, b_e2,
                                   W_e2, W_c1, b_c1, W_c2)
        aggH_k, aggT_k = _sc_scatter(row_k, EF, tx, ty, tz, zH)
        aggHs.append(aggH_k)
        aggTs.append(aggT_k)
    aggH = jnp.concatenate(aggHs, axis=0)
    aggT = jnp.concatenate(aggTs, axis=0)
    h_out, c_out4 = _node_update(h, coord4, aggH, aggT,
                                 W_n1, b_n1, W_n2, b_n2)
    return (h_out, c_out4[:, 0:3], edge_attr)
